# baseline (speedup 1.0000x reference)
import jax
import jax.numpy as jnp
from jax.experimental import pallas as pl
from jax.experimental.pallas import tpu as pltpu

_NE = 1024
_D = 64
_B = 16
_L = 1024
_ROWS = _B * _L
_BLK = 1024
_PER_STEP = 2
_GRID = _B // _PER_STEP


def _vq_body(xt_ref, w_ref, enc_ref, q_ref, loss_ref, perp_ref,
             counts_scr, sq_scr, bcol_scr, icol_scr, wt_scr):
    i = pl.program_id(0)

    @pl.when(i == 0)
    def _init():
        counts_scr[...] = jnp.zeros_like(counts_scr)
        sq_scr[0, 0] = 0.0
        w0 = w_ref[...]
        wt_scr[...] = w0.T
        bcol_scr[...] = jnp.sum(w0 * w0, axis=1)[:, None]
        icol_scr[...] = jax.lax.broadcasted_iota(
            jnp.int32, (_NE, 1), 0).astype(jnp.float32)

    w = w_ref[...]
    wt = wt_scr[...]
    bcol = bcol_scr[...]
    icol = icol_scr[...]
    iota_row = jax.lax.broadcasted_iota(jnp.int32, (1, _NE), 1).astype(
        jnp.float32)

    xts = [xt_ref[k] for k in range(_PER_STEP)]
    mTs = [jax.lax.dot_general(wt, xt, (((0,), (0,)), ((), ())),
                               preferred_element_type=jnp.float32)
           for xt in xts]

    csum = jnp.zeros((1, _NE), jnp.float32)
    ssum = 0.0
    for k in range(_PER_STEP):
        xt = xts[k]
        mT = mTs[k]
        a = jnp.sum(xt * xt, axis=0)[None, :]
        dT = a + bcol - 2.0 * mT

        dmin = jnp.min(dT, axis=0)
        idx = jnp.min(jnp.where(dT == dmin[None, :], icol, float(_NE)),
                      axis=0)
        enc = (iota_row == idx[:, None]).astype(jnp.float32)

        enc_ref[pl.ds(k * _BLK, _BLK), :] = enc
        q_ref[k] = jax.lax.dot_general(w, enc, (((0,), (1,)), ((), ())),
                                       preferred_element_type=jnp.float32)
        csum = csum + jnp.sum(enc, axis=0)[None, :]
        ssum = ssum + jnp.sum(dmin)

    counts_scr[...] += csum
    sq_scr[0, 0] += ssum

    @pl.when(i == _GRID - 1)
    def _fin():
        n_elems = float(_ROWS * _D)
        loss_ref[0, 0] = 1.25 * sq_scr[0, 0] / n_elems
        p = counts_scr[...] / float(_ROWS)
        ent = jnp.sum(p * jnp.log(p + 1e-10))
        perp_ref[0, 0] = jnp.exp(-ent)


def kernel(inputs, W):
    enc, q, loss, perp = pl.pallas_call(
        _vq_body,
        grid=(_GRID,),
        in_specs=[
            pl.BlockSpec((_PER_STEP, _D, _BLK), lambda i: (i, 0, 0)),
            pl.BlockSpec((_NE, _D), lambda i: (0, 0)),
        ],
        out_specs=[
            pl.BlockSpec((_PER_STEP * _BLK, _NE), lambda i: (i, 0)),
            pl.BlockSpec((_PER_STEP, _D, _BLK), lambda i: (i, 0, 0)),
            pl.BlockSpec(memory_space=pltpu.SMEM),
            pl.BlockSpec(memory_space=pltpu.SMEM),
        ],
        out_shape=[
            jax.ShapeDtypeStruct((_ROWS, _NE), jnp.float32),
            jax.ShapeDtypeStruct((_B, _D, _L), jnp.float32),
            jax.ShapeDtypeStruct((1, 1), jnp.float32),
            jax.ShapeDtypeStruct((1, 1), jnp.float32),
        ],
        scratch_shapes=[
            pltpu.VMEM((1, _NE), jnp.float32),
            pltpu.SMEM((1, 1), jnp.float32),
            pltpu.VMEM((_NE, 1), jnp.float32),
            pltpu.VMEM((_NE, 1), jnp.float32),
            pltpu.VMEM((_D, _NE), jnp.float32),
        ],
    )(inputs, W)
    return (loss[0, 0], q, perp[0, 0], enc)
